# baseline (speedup 1.0000x reference)
import jax
import jax.numpy as jnp
from jax.experimental import pallas as pl

_BM_SUPPORT = 1000
_BM_SPMM = 200


def _support_kernel(x_ref, w_ref, s_ref):
    x = x_ref[...].astype(jnp.bfloat16)
    w = w_ref[...].astype(jnp.bfloat16)
    acc = jnp.dot(x, w, preferred_element_type=jnp.float32)
    s_ref[...] = acc.astype(jnp.bfloat16)


def _spmm_relu_kernel(a_ref, s_ref, o_ref):
    a = a_ref[...].astype(jnp.bfloat16)
    acc = jnp.dot(a, s_ref[...], preferred_element_type=jnp.float32)
    o_ref[...] = jnp.maximum(acc, 0.0)


def kernel(features, adj, weight):
    n, d_in = features.shape
    d_out = weight.shape[1]

    support = pl.pallas_call(
        _support_kernel,
        grid=(n // _BM_SUPPORT,),
        in_specs=[
            pl.BlockSpec((_BM_SUPPORT, d_in), lambda i: (i, 0)),
            pl.BlockSpec((d_in, d_out), lambda i: (0, 0)),
        ],
        out_specs=pl.BlockSpec((_BM_SUPPORT, d_out), lambda i: (i, 0)),
        out_shape=jax.ShapeDtypeStruct((n, d_out), jnp.bfloat16),
    )(features, weight)

    out = pl.pallas_call(
        _spmm_relu_kernel,
        grid=(n // _BM_SPMM,),
        in_specs=[
            pl.BlockSpec((_BM_SPMM, n), lambda i: (i, 0)),
            pl.BlockSpec((n, d_out), lambda i: (0, 0)),
        ],
        out_specs=pl.BlockSpec((_BM_SPMM, d_out), lambda i: (i, 0)),
        out_shape=jax.ShapeDtypeStruct((n, d_out), jnp.float32),
    )(adj, support)

    return out
